# baseline (speedup 1.0000x reference)
import jax
import jax.numpy as jnp
from jax import lax
from jax.experimental import pallas as pl
from jax.experimental.pallas import tpu as pltpu
from jax.experimental.pallas import tpu_sc as plsc

B = 16384
D = 64
NC = 2
NS = 16
L = 16
NW = NC * NS
BPW = B // NW
CH = 64
NCH = BPW // CH
NPH = 3 * NCH
GPC = CH // L
ENT_R = 500000
REL_R = 500
MARGIN = 1.0
UNROLL = 16


def _rsqrt_lanes(x):
    x = jnp.maximum(x, 1e-12)
    i = plsc.bitcast(x, jnp.int32)
    y = plsc.bitcast(jnp.int32(0x5F3759DF) - (i >> 1), jnp.float32)
    for _ in range(3):
        y = y * (1.5 - 0.5 * x * y * y)
    return y


def _transe_body(ent_hbm, rel_hbm,
                 ph_hbm, pt_hbm, pr_hbm,
                 nh_hbm, nt_hbm, nr_hbm,
                 qh_hbm, qt_hbm, qr_hbm,
                 p_out, n_out, pred_out, loss_out,
                 rel_v, idx_v,
                 h_a, t_a, h_b, t_b, h_c, t_c,
                 out_v, loss_v, sem_a, sem_b, sem_c):
    wid = lax.axis_index("s") * NC + lax.axis_index("c")
    base = wid * BPW
    lane = lax.iota(jnp.int32, L)
    zero = jnp.zeros((L,), jnp.float32)
    izero = jnp.zeros((L,), jnp.int32)

    pltpu.sync_copy(rel_hbm, rel_v)

    idx_srcs = (ph_hbm, pt_hbm, pr_hbm, nh_hbm, nt_hbm, nr_hbm,
                qh_hbm, qt_hbm, qr_hbm)
    for a, src in enumerate(idx_srcs):
        pltpu.sync_copy(src.at[pl.ds(base, BPW)],
                        idx_v.at[pl.ds(a * BPW, BPW)])

    bufs = ((h_a, t_a), (h_b, t_b), (h_c, t_c))

    def fire(p, parity, psem):
        tr = p // NCH
        c = p % NCH
        hb, tb = bufs[parity]
        for g in range(GPC):
            hvec = idx_v[pl.ds(3 * tr * BPW + c * CH + g * L, L)]
            tvec = idx_v[pl.ds((3 * tr + 1) * BPW + c * CH + g * L, L)]
            for u in range(L):
                j = g * L + u
                pltpu.async_copy(ent_hbm.at[pl.ds(hvec[u], 1)],
                                 hb.at[pl.ds(j, 1)], psem)
                pltpu.async_copy(ent_hbm.at[pl.ds(tvec[u], 1)],
                                 tb.at[pl.ds(j, 1)], psem)

    def drain(parity, psem):
        hb, tb = bufs[parity]
        pltpu.make_async_copy(ent_hbm.at[pl.ds(0, CH)], hb, psem).wait()
        pltpu.make_async_copy(ent_hbm.at[pl.ds(0, CH)], tb, psem).wait()

    def compute(p, parity):
        tr = p // NCH
        c = p % NCH
        hb, tb = bufs[parity]
        scale = jnp.where(tr == 2, jnp.float32(1.0 / D), jnp.float32(1.0))

        def group(g, _):
            gbase = c * CH + g * L
            rows = g * L + lane
            ridx = idx_v[pl.ds((3 * tr + 2) * BPW + gbase, L)]
            rrow = ridx >> 1
            rful = (ridx & 1) * D

            def p1(db, accs):
                ah, at2, ar = accs
                dd = lane + db * UNROLL
                for u in range(UNROLL):
                    du = (dd + u) & (D - 1)
                    hv = plsc.load_gather(hb, [rows, du])
                    tv = plsc.load_gather(tb, [rows, du])
                    rv = plsc.load_gather(rel_v, [rrow, rful + du])
                    ah = ah + hv * hv
                    at2 = at2 + tv * tv
                    ar = ar + rv * rv
                return (ah, at2, ar)

            ah, at2, ar = lax.fori_loop(0, D // UNROLL, p1,
                                        (zero, zero, zero))
            ih = _rsqrt_lanes(ah)
            it2 = _rsqrt_lanes(at2)
            ir = _rsqrt_lanes(ar)

            def p2(db, acc):
                dd = lane + db * UNROLL
                for u in range(UNROLL):
                    du = (dd + u) & (D - 1)
                    hv = plsc.load_gather(hb, [rows, du])
                    tv = plsc.load_gather(tb, [rows, du])
                    rv = plsc.load_gather(rel_v, [rrow, rful + du])
                    acc = acc + jnp.abs(hv * ih + rv * ir - tv * it2)
                return acc

            s = lax.fori_loop(0, D // UNROLL, p2, zero)
            out_v[pl.ds(tr * BPW + gbase, L)] = s * scale
            return 0

        lax.fori_loop(0, GPC, group, 0)

    sems = (sem_a, sem_b, sem_c)
    fire(0, 0, sem_a)
    fire(1, 1, sem_b)

    def round_body(rnd, _):
        for b in (0, 1, 2):
            p = rnd * 3 + b
            nb = (b + 2) % 3
            fire((p + 2) % NPH, nb, sems[nb])
            drain(b, sems[b])
            compute(p, b)
        return 0

    lax.fori_loop(0, NPH // 3, round_body, 0)
    drain(0, sem_a)
    drain(1, sem_b)

    def lgroup(g, acc):
        lp = out_v[pl.ds(g * L, L)]
        ln = out_v[pl.ds(BPW + g * L, L)]
        return acc + jnp.maximum(lp - ln + MARGIN, 0.0)

    lacc = lax.fori_loop(0, BPW // L, lgroup, zero)
    loss_v[...] = lacc * (1.0 / B)

    out_slice = pl.ds(base, BPW)
    pltpu.sync_copy(out_v.at[pl.ds(0, BPW)], p_out.at[out_slice])
    pltpu.sync_copy(out_v.at[pl.ds(BPW, BPW)], n_out.at[out_slice])
    pltpu.sync_copy(out_v.at[pl.ds(2 * BPW, BPW)], pred_out.at[out_slice])
    pltpu.sync_copy(loss_v, loss_out.at[wid])


@jax.jit
def kernel(ent_embeddings, rel_embeddings, pos_h, pos_t, pos_r,
           neg_h, neg_t, neg_r, h, t, r):
    mesh = plsc.VectorSubcoreMesh(core_axis_name="c", subcore_axis_name="s",
                                  num_cores=NC, num_subcores=NS)
    f = pl.kernel(
        _transe_body,
        out_type=(
            jax.ShapeDtypeStruct((B,), jnp.float32),
            jax.ShapeDtypeStruct((B,), jnp.float32),
            jax.ShapeDtypeStruct((B,), jnp.float32),
            jax.ShapeDtypeStruct((NW, L), jnp.float32),
        ),
        mesh=mesh,
        compiler_params=pltpu.CompilerParams(needs_layout_passes=False),
        scratch_types=[
            pltpu.VMEM((REL_R, 2 * D), jnp.float32),
            pltpu.VMEM((9 * BPW,), jnp.int32),
            pltpu.VMEM((CH, D), jnp.float32),
            pltpu.VMEM((CH, D), jnp.float32),
            pltpu.VMEM((CH, D), jnp.float32),
            pltpu.VMEM((CH, D), jnp.float32),
            pltpu.VMEM((CH, D), jnp.float32),
            pltpu.VMEM((CH, D), jnp.float32),
            pltpu.VMEM((3 * BPW,), jnp.float32),
            pltpu.VMEM((L,), jnp.float32),
            pltpu.SemaphoreType.DMA,
            pltpu.SemaphoreType.DMA,
            pltpu.SemaphoreType.DMA,
        ],
    )
    rel2 = jnp.reshape(rel_embeddings, (REL_R, 2 * D))
    p, n, pred, lparts = f(ent_embeddings, rel2,
                           pos_h, pos_t, pos_r,
                           neg_h, neg_t, neg_r, h, t, r)
    return p[:, None], n[:, None], pred, jnp.sum(lparts)


# speedup vs baseline: 1.0365x; 1.0365x over previous
import jax
import jax.numpy as jnp
from jax import lax
from jax.experimental import pallas as pl
from jax.experimental.pallas import tpu as pltpu
from jax.experimental.pallas import tpu_sc as plsc

B = 16384
D = 64
NC = 2
NS = 16
L = 16
NW = NC * NS
BPW = B // NW
CH = 64
NCH = BPW // CH
NPH = 3 * NCH
GPC = CH // L
ENT_R = 500000
REL_R = 500
MARGIN = 1.0
UNROLL = 16


def _rsqrt_lanes(x):
    x = jnp.maximum(x, 1e-12)
    i = plsc.bitcast(x, jnp.int32)
    y = plsc.bitcast(jnp.int32(0x5F3759DF) - (i >> 1), jnp.float32)
    for _ in range(3):
        y = y * (1.5 - 0.5 * x * y * y)
    return y


def _transe_body(ent_hbm, rel_hbm,
                 ph_hbm, pt_hbm, pr_hbm,
                 nh_hbm, nt_hbm, nr_hbm,
                 qh_hbm, qt_hbm, qr_hbm,
                 p_out, n_out, pred_out, loss_out,
                 rel_v, idx_v,
                 h_a, t_a, h_b, t_b,
                 out_v, loss_v, sem_a, sem_b):
    wid = lax.axis_index("s") * NC + lax.axis_index("c")
    base = wid * BPW
    lane = lax.iota(jnp.int32, L)
    zero = jnp.zeros((L,), jnp.float32)
    izero = jnp.zeros((L,), jnp.int32)

    pltpu.sync_copy(rel_hbm, rel_v)

    idx_srcs = (ph_hbm, pt_hbm, pr_hbm, nh_hbm, nt_hbm, nr_hbm,
                qh_hbm, qt_hbm, qr_hbm)
    idx_handles = [
        pltpu.async_copy(src.at[pl.ds(base, BPW)],
                         idx_v.at[pl.ds(a * BPW, BPW)], sem_b)
        for a, src in enumerate(idx_srcs)
    ]
    for hd in idx_handles:
        hd.wait()

    bufs = ((h_a, t_a), (h_b, t_b))

    def fire(p, parity, psem):
        tr = p // NCH
        c = p % NCH
        hb, tb = bufs[parity]
        for g in range(GPC):
            hvec = idx_v[pl.ds(3 * tr * BPW + c * CH + g * L, L)]
            tvec = idx_v[pl.ds((3 * tr + 1) * BPW + c * CH + g * L, L)]
            for u in range(L):
                j = g * L + u
                pltpu.async_copy(ent_hbm.at[pl.ds(hvec[u], 1)],
                                 hb.at[pl.ds(j, 1)], psem)
                pltpu.async_copy(ent_hbm.at[pl.ds(tvec[u], 1)],
                                 tb.at[pl.ds(j, 1)], psem)

    def drain(parity, psem):
        hb, tb = bufs[parity]
        pltpu.make_async_copy(ent_hbm.at[pl.ds(0, CH)], hb, psem).wait()
        pltpu.make_async_copy(ent_hbm.at[pl.ds(0, CH)], tb, psem).wait()

    def compute(p, parity):
        tr = p // NCH
        c = p % NCH
        hb, tb = bufs[parity]
        scale = jnp.where(tr == 2, jnp.float32(1.0 / D), jnp.float32(1.0))

        def group(g, _):
            gbase = c * CH + g * L
            rows = g * L + lane
            ridx = idx_v[pl.ds((3 * tr + 2) * BPW + gbase, L)]
            rrow = ridx >> 1
            rful = (ridx & 1) * D

            def p1(db, accs):
                ah, at2, ar = accs
                dd = lane + db * UNROLL
                for u in range(UNROLL):
                    du = (dd + u) & (D - 1)
                    hv = plsc.load_gather(hb, [rows, du])
                    tv = plsc.load_gather(tb, [rows, du])
                    rv = plsc.load_gather(rel_v, [rrow, rful + du])
                    ah = ah + hv * hv
                    at2 = at2 + tv * tv
                    ar = ar + rv * rv
                return (ah, at2, ar)

            ah, at2, ar = lax.fori_loop(0, D // UNROLL, p1,
                                        (zero, zero, zero))
            ih = _rsqrt_lanes(ah)
            it2 = _rsqrt_lanes(at2)
            ir = _rsqrt_lanes(ar)

            def p2(db, acc):
                dd = lane + db * UNROLL
                for u in range(UNROLL):
                    du = (dd + u) & (D - 1)
                    hv = plsc.load_gather(hb, [rows, du])
                    tv = plsc.load_gather(tb, [rows, du])
                    rv = plsc.load_gather(rel_v, [rrow, rful + du])
                    acc = acc + jnp.abs(hv * ih + rv * ir - tv * it2)
                return acc

            s = lax.fori_loop(0, D // UNROLL, p2, zero)
            out_v[pl.ds(tr * BPW + gbase, L)] = s * scale
            return 0

        lax.fori_loop(0, GPC, group, 0)

    fire(0, 0, sem_a)
    sems = (sem_a, sem_b)

    def round_body(rnd, _):
        for parity in (0, 1):
            p = rnd * 2 + parity
            fire((p + 1) % NPH, 1 - parity, sems[1 - parity])
            drain(parity, sems[parity])
            compute(p, parity)
        return 0

    lax.fori_loop(0, NPH // 2, round_body, 0)
    drain(0, sem_a)

    def lgroup(g, acc):
        lp = out_v[pl.ds(g * L, L)]
        ln = out_v[pl.ds(BPW + g * L, L)]
        return acc + jnp.maximum(lp - ln + MARGIN, 0.0)

    lacc = lax.fori_loop(0, BPW // L, lgroup, zero)
    loss_v[...] = lacc * (1.0 / B)

    out_slice = pl.ds(base, BPW)
    pltpu.sync_copy(out_v.at[pl.ds(0, BPW)], p_out.at[out_slice])
    pltpu.sync_copy(out_v.at[pl.ds(BPW, BPW)], n_out.at[out_slice])
    pltpu.sync_copy(out_v.at[pl.ds(2 * BPW, BPW)], pred_out.at[out_slice])
    pltpu.sync_copy(loss_v, loss_out.at[wid])


@jax.jit
def kernel(ent_embeddings, rel_embeddings, pos_h, pos_t, pos_r,
           neg_h, neg_t, neg_r, h, t, r):
    mesh = plsc.VectorSubcoreMesh(core_axis_name="c", subcore_axis_name="s",
                                  num_cores=NC, num_subcores=NS)
    f = pl.kernel(
        _transe_body,
        out_type=(
            jax.ShapeDtypeStruct((B,), jnp.float32),
            jax.ShapeDtypeStruct((B,), jnp.float32),
            jax.ShapeDtypeStruct((B,), jnp.float32),
            jax.ShapeDtypeStruct((NW, L), jnp.float32),
        ),
        mesh=mesh,
        compiler_params=pltpu.CompilerParams(needs_layout_passes=False),
        scratch_types=[
            pltpu.VMEM((REL_R, 2 * D), jnp.float32),
            pltpu.VMEM((9 * BPW,), jnp.int32),
            pltpu.VMEM((CH, D), jnp.float32),
            pltpu.VMEM((CH, D), jnp.float32),
            pltpu.VMEM((CH, D), jnp.float32),
            pltpu.VMEM((CH, D), jnp.float32),
            pltpu.VMEM((3 * BPW,), jnp.float32),
            pltpu.VMEM((L,), jnp.float32),
            pltpu.SemaphoreType.DMA,
            pltpu.SemaphoreType.DMA,
        ],
    )
    rel2 = jnp.reshape(rel_embeddings, (REL_R, 2 * D))
    p, n, pred, lparts = f(ent_embeddings, rel2,
                           pos_h, pos_t, pos_r,
                           neg_h, neg_t, neg_r, h, t, r)
    return p[:, None], n[:, None], pred, jnp.sum(lparts)
